# Initial kernel scaffold; baseline (speedup 1.0000x reference)
import jax
import jax.numpy as jnp
from jax.experimental import pallas as pl


def kernel(v, edges, W1, b1, W2, b2, W3, b3, Wc, bc, gn1_alpha, gn1_weight, gn1_bias, gn2_alpha, gn2_weight, gn2_bias, gn3_alpha, gn3_weight, gn3_bias):
    raise NotImplementedError("write your pallas kernel here")


# speedup vs baseline: 4.2107x; 4.2107x over previous
import functools

import jax
import jax.numpy as jnp
from jax import lax
from jax.experimental import pallas as pl
from jax.experimental.pallas import tpu as pltpu
from jax.experimental.pallas import tpu_sc as plsc

N = 10000
E = 160000
N_PAD = 10240
E_PAD = 163840
DUMMY_DST = 10008
F32 = jnp.float32

_MESH = functools.partial(
    plsc.VectorSubcoreMesh,
    core_axis_name="c",
    subcore_axis_name="s",
    num_cores=2,
    num_subcores=16,
)
STRIPE = N_PAD // 16


def _build_hist():
    nch = E_PAD // (16 * 64)

    @functools.partial(
        pl.kernel,
        out_type=jax.ShapeDtypeStruct((N_PAD,), F32),
        mesh=_MESH(),
        compiler_params=pltpu.CompilerParams(needs_layout_passes=False),
        scratch_types=[
            pltpu.VMEM((nch, 64), jnp.int32),
            pltpu.VMEM((N_PAD,), F32),
            pltpu.VMEM((16, STRIPE), F32),
            pltpu.VMEM((STRIPE,), F32),
            pltpu.VMEM_SHARED((16, N_PAD), F32),
        ],
    )
    def hist(dst_hbm, out_hbm, idx_v, hist_v, loc2, degl, sh):
        s = lax.axis_index("s")
        pltpu.sync_copy(dst_hbm.at[s], idx_v)

        def _zero(i, _):
            hist_v[pl.ds(i * 16, 16)] = jnp.zeros((16,), F32)
            return 0

        lax.fori_loop(0, N_PAD // 16, _zero, 0)
        ones = jnp.ones((16,), F32)

        def _acc(i, _):
            idx = idx_v[i // 4, pl.ds((i % 4) * 16, 16)]
            plsc.addupdate_scatter(hist_v, [idx], ones)
            return 0

        lax.fori_loop(0, (nch * 64) // 16, _acc, 0)
        pltpu.sync_copy(hist_v, sh.at[s])
        plsc.subcore_barrier()
        for r in range(16):
            pltpu.sync_copy(sh.at[r, pl.ds(s * STRIPE, STRIPE)], loc2.at[r])

        def _merge(j, _):
            sl = pl.ds(j * 16, 16)
            a = loc2[0, sl]
            for r in range(1, 16):
                a = a + loc2[r, sl]
            degl[sl] = a + 1.0
            return 0

        lax.fori_loop(0, STRIPE // 16, _merge, 0)
        pltpu.sync_copy(degl, out_hbm.at[pl.ds(s * STRIPE, STRIPE)])

    return hist


def _build_agg(nb, w, split):
    ntile = 32 if split else 16
    nch = E_PAD // (ntile * 64)
    nch_h = nch // 2
    npairs = nch_h // 2 - 1
    nb_out = 2 if split else nb
    blocks_per_core = 1 if split else nb // 2

    @functools.partial(
        pl.kernel,
        out_type=jax.ShapeDtypeStruct((nb_out, N_PAD, w), F32),
        mesh=_MESH(),
        compiler_params=pltpu.CompilerParams(needs_layout_passes=False),
        scratch_types=[
            pltpu.VMEM((nch_h, 64), jnp.int32),
            pltpu.VMEM((nch_h, 64), jnp.int32),
            pltpu.VMEM((64, w), F32),
            pltpu.VMEM((64, w), F32),
            pltpu.VMEM_SHARED((N_PAD, w), F32),
            pltpu.SemaphoreType.DMA,
            pltpu.SemaphoreType.DMA,
        ],
    )
    def agg(s_hbm, src_hbm, dst_hbm, out_hbm, src_v, dst_v, buf_a, buf_b,
            acc, sem_a, sem_b):
        c = lax.axis_index("c")
        s = lax.axis_index("s")
        t = s * 2 + c if split else s

        def _phase(cb, out_i):
            def _zero(i, _):
                buf_a[i // (w // 16), pl.ds((i % (w // 16)) * 16, 16)] = (
                    jnp.zeros((16,), F32))
                buf_b[i // (w // 16), pl.ds((i % (w // 16)) * 16, 16)] = (
                    jnp.zeros((16,), F32))
                return 0

            lax.fori_loop(0, (64 * w) // 16, _zero, 0)
            for kz in range(STRIPE // 64):
                pltpu.sync_copy(buf_a, acc.at[pl.ds(s * STRIPE + kz * 64, 64)])
            plsc.subcore_barrier()
            s_view = s_hbm.at[cb]

            def _scat(j, buf):
                pltpu.sync_copy(buf, acc.at[dst_v.at[j]], add=True)

            for half in range(2):
                pltpu.sync_copy(src_hbm.at[t, pl.ds(half * nch_h, nch_h)],
                                src_v)
                pltpu.sync_copy(dst_hbm.at[t, pl.ds(half * nch_h, nch_h)],
                                dst_v)

                def _pair(i, _):
                    j0 = 2 * i
                    da = pltpu.async_copy(s_view.at[src_v.at[j0]], buf_a,
                                          sem_a)
                    db = pltpu.async_copy(s_view.at[src_v.at[j0 + 1]], buf_b,
                                          sem_b)
                    da.wait()
                    _scat(j0, buf_a)
                    db.wait()
                    _scat(j0 + 1, buf_b)
                    return 0

                lax.fori_loop(0, nch_h // 2, _pair, 0)
            plsc.subcore_barrier()
            pltpu.sync_copy(
                acc.at[pl.ds(s * STRIPE, STRIPE)],
                out_hbm.at[out_i].at[pl.ds(s * STRIPE, STRIPE)],
            )

        if split:
            _phase(0, c)
        else:
            for k in range(blocks_per_core):
                cb = c + 2 * k
                _phase(cb, cb)

    return agg


_build_hist_cached = functools.cache(_build_hist)
_build_agg_cached = functools.cache(_build_agg)


def _sc_hist(d16):
    return _build_hist_cached()(d16)


def _sc_agg(nb, w, split, s_blk, src_i, dst_i):
    return _build_agg_cached(nb, w, split)(s_blk, src_i, dst_i)


def _scale_call(v_pad, deg_c):
    br = 512

    def body(v_ref, deg_ref, out_ref):
        dinv = lax.rsqrt(deg_ref[...])
        out_ref[0] = v_ref[...] * dinv

    return pl.pallas_call(
        body,
        grid=(2, N_PAD // br),
        in_specs=[
            pl.BlockSpec((br, 128), lambda cb, r: (r, cb)),
            pl.BlockSpec((br, 1), lambda cb, r: (r, 0)),
        ],
        out_specs=pl.BlockSpec((1, br, 128), lambda cb, r: (cb, r, 0)),
        out_shape=jax.ShapeDtypeStruct((2, N_PAD, 128), F32),
    )(v_pad, deg_c)


def _b_call(e_blk, x, deg_c, w, b_row, relu=True):
    nbk = x.shape[1] // 128
    f_out = w.shape[1]
    br = 256
    bc = min(512, f_out)
    nc = f_out // bc

    def body(e_ref, x_ref, deg_ref, w_ref, b_ref, h_ref, s_ref):
        r = pl.program_id(1)
        deg = deg_ref[...]
        dinv = lax.rsqrt(deg)
        dinv2 = 1.0 / deg
        acc = jnp.zeros((br, bc), F32)
        for k in range(nbk):
            m = dinv * e_ref[k] + dinv2 * x_ref[:, k * 128:(k + 1) * 128]
            acc = acc + lax.dot_general(
                m, w_ref[pl.ds(k * 128, 128), :],
                (((1,), (0,)), ((), ())),
                preferred_element_type=F32)
        z = acc + b_ref[...]
        h = jnp.maximum(z, 0.0) if relu else z
        h_ref[...] = h
        rows = r * br + lax.broadcasted_iota(jnp.int32, (br, bc), 0)
        hm = jnp.where(rows < N, h, 0.0)
        s1 = jnp.sum(hm, axis=0, keepdims=True)
        s2 = jnp.sum(hm * hm, axis=0, keepdims=True)
        sblk = jnp.concatenate([s1, s2, jnp.zeros((6, bc), F32)], axis=0)

        @pl.when(r == 0)
        def _():
            s_ref[...] = sblk

        @pl.when(r > 0)
        def _():
            s_ref[...] = s_ref[...] + sblk

    return pl.pallas_call(
        body,
        grid=(nc, N_PAD // br),
        in_specs=[
            pl.BlockSpec((nbk, br, 128), lambda c, r: (0, r, 0)),
            pl.BlockSpec((br, nbk * 128), lambda c, r: (r, 0)),
            pl.BlockSpec((br, 1), lambda c, r: (r, 0)),
            pl.BlockSpec((nbk * 128, bc), lambda c, r: (0, c)),
            pl.BlockSpec((1, bc), lambda c, r: (0, c)),
        ],
        out_specs=[
            pl.BlockSpec((br, bc), lambda c, r: (r, c)),
            pl.BlockSpec((8, bc), lambda c, r: (0, c)),
        ],
        out_shape=[
            jax.ShapeDtypeStruct((N_PAD, f_out), F32),
            jax.ShapeDtypeStruct((8, f_out), F32),
        ],
    )(e_blk, x, deg_c, w, b_row)


def _c_call(h, s, deg_c, a_row, g_row, be_row, emit_s=True):
    f = h.shape[1]
    br = 512

    def body(h_ref, s_ref, deg_ref, a_ref, g_ref, be_ref, *outs):
        s1 = s_ref[0:1, :]
        s2 = s_ref[1:2, :]
        al = a_ref[...]
        m = s1 * (1.0 / N)
        var = s2 * (1.0 / N) - m * m * al * (2.0 - al)
        inv = lax.rsqrt(var + 1e-5)
        xn = (h_ref[...] - al * m) * inv * g_ref[...] + be_ref[...]
        outs[0][...] = xn
        if emit_s:
            outs[1][0] = xn * lax.rsqrt(deg_ref[...])

    out_specs = [pl.BlockSpec((br, 128), lambda c, r: (r, c))]
    out_shape = [jax.ShapeDtypeStruct((N_PAD, f), F32)]
    if emit_s:
        out_specs.append(pl.BlockSpec((1, br, 128), lambda c, r: (c, r, 0)))
        out_shape.append(jax.ShapeDtypeStruct((f // 128, N_PAD, 128), F32))

    res = pl.pallas_call(
        body,
        grid=(f // 128, N_PAD // br),
        in_specs=[
            pl.BlockSpec((br, 128), lambda c, r: (r, c)),
            pl.BlockSpec((8, 128), lambda c, r: (0, c)),
            pl.BlockSpec((br, 1), lambda c, r: (r, 0)),
            pl.BlockSpec((1, 128), lambda c, r: (0, c)),
            pl.BlockSpec((1, 128), lambda c, r: (0, c)),
            pl.BlockSpec((1, 128), lambda c, r: (0, c)),
        ],
        out_specs=out_specs,
        out_shape=out_shape,
    )(h, s, deg_c, a_row, g_row, be_row)
    return res if emit_s else (res[0],)


def _b4_call(x3, deg_c, wc):
    nbk = x3.shape[1] // 128
    br = 256

    def body(x_ref, deg_ref, w_ref, z_ref, s4_ref):
        acc = jnp.zeros((br, 64), F32)
        for k in range(nbk):
            acc = acc + lax.dot_general(
                x_ref[:, k * 128:(k + 1) * 128],
                w_ref[pl.ds(k * 128, 128), :],
                (((1,), (0,)), ((), ())),
                preferred_element_type=F32)
        z_ref[...] = acc
        s4_ref[0] = jnp.concatenate(
            [acc * lax.rsqrt(deg_ref[...]), jnp.zeros((br, 64), F32)], axis=1)

    return pl.pallas_call(
        body,
        grid=(N_PAD // br,),
        in_specs=[
            pl.BlockSpec((br, nbk * 128), lambda r: (r, 0)),
            pl.BlockSpec((br, 1), lambda r: (r, 0)),
            pl.BlockSpec((nbk * 128, 64), lambda r: (0, 0)),
        ],
        out_specs=[
            pl.BlockSpec((br, 64), lambda r: (r, 0)),
            pl.BlockSpec((1, br, 128), lambda r: (0, r, 0)),
        ],
        out_shape=[
            jax.ShapeDtypeStruct((N_PAD, 64), F32),
            jax.ShapeDtypeStruct((1, N_PAD, 128), F32),
        ],
    )(x3, deg_c, wc)


def _d_call(e4, z4, deg_c, bc_row):
    br = 400

    def body(e_ref, z_ref, deg_ref, b_ref, lg_ref, sm_ref):
        deg = deg_ref[...]
        dinv = lax.rsqrt(deg)
        dinv2 = 1.0 / deg
        lg = (dinv * (e_ref[0, :, :64] + e_ref[1, :, :64])
              + dinv2 * z_ref[...] + b_ref[...])
        lg_ref[...] = lg
        mx = jnp.max(lg, axis=1, keepdims=True)
        p = jnp.exp(lg - mx)
        sm_ref[...] = p / jnp.sum(p, axis=1, keepdims=True)

    return pl.pallas_call(
        body,
        grid=(N // br,),
        in_specs=[
            pl.BlockSpec((2, br, 128), lambda r: (0, r, 0)),
            pl.BlockSpec((br, 64), lambda r: (r, 0)),
            pl.BlockSpec((br, 1), lambda r: (r, 0)),
            pl.BlockSpec((1, 64), lambda r: (0, 0)),
        ],
        out_specs=[
            pl.BlockSpec((br, 64), lambda r: (r, 0)),
            pl.BlockSpec((br, 64), lambda r: (r, 0)),
        ],
        out_shape=[
            jax.ShapeDtypeStruct((N, 64), F32),
            jax.ShapeDtypeStruct((N, 64), F32),
        ],
    )(e4, z4, deg_c, bc_row)


def kernel(v, edges, W1, b1, W2, b2, W3, b3, Wc, bc,
           gn1_alpha, gn1_weight, gn1_bias,
           gn2_alpha, gn2_weight, gn2_bias,
           gn3_alpha, gn3_weight, gn3_bias):
    src = edges[0].astype(jnp.int32)
    dst = edges[1].astype(jnp.int32)
    src_p = jnp.concatenate([src, jnp.zeros((E_PAD - E,), jnp.int32)])
    dst_p = jnp.concatenate([dst, jnp.full((E_PAD - E,), DUMMY_DST, jnp.int32)])
    s16 = src_p.reshape(16, E_PAD // (16 * 64), 64)
    d16 = dst_p.reshape(16, E_PAD // (16 * 64), 64)
    s32 = src_p.reshape(32, E_PAD // (32 * 64), 64)
    d32 = dst_p.reshape(32, E_PAD // (32 * 64), 64)
    v_pad = jnp.pad(v, ((0, N_PAD - N), (0, 0)))

    deg = _sc_hist(d16)
    deg_c = deg.reshape(N_PAD, 1)

    s0 = _scale_call(v_pad, deg_c)
    e1 = _sc_agg(2, 128, False, s0, s16, d16)
    h1, sums1 = _b_call(e1, v_pad, deg_c, W1, b1.reshape(1, -1))
    x1, s1 = _c_call(h1, sums1, deg_c, gn1_alpha.reshape(1, -1),
                     gn1_weight.reshape(1, -1), gn1_bias.reshape(1, -1))

    e2 = _sc_agg(4, 128, False, s1, s16, d16)
    h2, sums2 = _b_call(e2, x1, deg_c, W2, b2.reshape(1, -1))
    x2, s2 = _c_call(h2, sums2, deg_c, gn2_alpha.reshape(1, -1),
                     gn2_weight.reshape(1, -1), gn2_bias.reshape(1, -1))

    e3 = _sc_agg(8, 128, False, s2, s16, d16)
    h3, sums3 = _b_call(e3, x2, deg_c, W3, b3.reshape(1, -1))
    (x3,) = _c_call(h3, sums3, deg_c, gn3_alpha.reshape(1, -1),
                    gn3_weight.reshape(1, -1), gn3_bias.reshape(1, -1),
                    emit_s=False)

    z4, s4 = _b4_call(x3, deg_c, Wc)
    e4 = _sc_agg(1, 128, True, s4, s32, d32)
    logits, sm = _d_call(e4, z4, deg_c, bc.reshape(1, -1))
    return (logits, sm)


# speedup vs baseline: 4.2843x; 1.0175x over previous
import functools

import jax
import jax.numpy as jnp
from jax import lax
from jax.experimental import pallas as pl
from jax.experimental.pallas import tpu as pltpu
from jax.experimental.pallas import tpu_sc as plsc

N = 10000
E = 160000
N_PAD = 10240
E_PAD = 163840
DUMMY_DST = 10008
F32 = jnp.float32

_MESH = functools.partial(
    plsc.VectorSubcoreMesh,
    core_axis_name="c",
    subcore_axis_name="s",
    num_cores=2,
    num_subcores=16,
)
STRIPE = N_PAD // 16


def _build_hist():
    nch = E_PAD // (16 * 64)

    @functools.partial(
        pl.kernel,
        out_type=jax.ShapeDtypeStruct((N_PAD,), F32),
        mesh=_MESH(),
        compiler_params=pltpu.CompilerParams(needs_layout_passes=False),
        scratch_types=[
            pltpu.VMEM((nch, 64), jnp.int32),
            pltpu.VMEM((N_PAD,), F32),
            pltpu.VMEM((16, STRIPE), F32),
            pltpu.VMEM((STRIPE,), F32),
            pltpu.VMEM_SHARED((16, N_PAD), F32),
        ],
    )
    def hist(dst_hbm, out_hbm, idx_v, hist_v, loc2, degl, sh):
        s = lax.axis_index("s")
        pltpu.sync_copy(dst_hbm.at[s], idx_v)

        def _zero(i, _):
            hist_v[pl.ds(i * 16, 16)] = jnp.zeros((16,), F32)
            return 0

        lax.fori_loop(0, N_PAD // 16, _zero, 0)
        ones = jnp.ones((16,), F32)

        def _acc(i, _):
            idx = idx_v[i // 4, pl.ds((i % 4) * 16, 16)]
            plsc.addupdate_scatter(hist_v, [idx], ones)
            return 0

        lax.fori_loop(0, (nch * 64) // 16, _acc, 0)
        pltpu.sync_copy(hist_v, sh.at[s])
        plsc.subcore_barrier()
        for r in range(16):
            pltpu.sync_copy(sh.at[r, pl.ds(s * STRIPE, STRIPE)], loc2.at[r])

        def _merge(j, _):
            sl = pl.ds(j * 16, 16)
            a = loc2[0, sl]
            for r in range(1, 16):
                a = a + loc2[r, sl]
            degl[sl] = a + 1.0
            return 0

        lax.fori_loop(0, STRIPE // 16, _merge, 0)
        pltpu.sync_copy(degl, out_hbm.at[pl.ds(s * STRIPE, STRIPE)])

    return hist


def _build_agg(nb, w, split):
    ntile = 32 if split else 16
    nch = E_PAD // (ntile * 64)
    nch_h = nch // 2
    npairs = nch_h // 2 - 1
    nb_out = 2 if split else nb
    blocks_per_core = 1 if split else nb // 2

    @functools.partial(
        pl.kernel,
        out_type=jax.ShapeDtypeStruct((nb_out, N_PAD, w), F32),
        mesh=_MESH(),
        compiler_params=pltpu.CompilerParams(needs_layout_passes=False),
        scratch_types=[
            pltpu.VMEM((nch_h, 64), jnp.int32),
            pltpu.VMEM((nch_h, 64), jnp.int32),
            pltpu.VMEM((64, w), F32),
            pltpu.VMEM((64, w), F32),
            pltpu.VMEM_SHARED((N_PAD, w), F32),
            pltpu.SemaphoreType.DMA,
            pltpu.SemaphoreType.DMA,
            pltpu.SemaphoreType.DMA,
            pltpu.SemaphoreType.DMA,
        ],
    )
    def agg(s_hbm, src_hbm, dst_hbm, out_hbm, src_v, dst_v, buf_a, buf_b,
            acc, sem_ga, sem_gb, sem_sa, sem_sb):
        c = lax.axis_index("c")
        s = lax.axis_index("s")
        t = s * 2 + c if split else s

        def _phase(cb, out_i):
            def _zero(i, _):
                buf_a[i // (w // 16), pl.ds((i % (w // 16)) * 16, 16)] = (
                    jnp.zeros((16,), F32))
                buf_b[i // (w // 16), pl.ds((i % (w // 16)) * 16, 16)] = (
                    jnp.zeros((16,), F32))
                return 0

            lax.fori_loop(0, (64 * w) // 16, _zero, 0)
            for kz in range(STRIPE // 64):
                pltpu.sync_copy(buf_a, acc.at[pl.ds(s * STRIPE + kz * 64, 64)])
            plsc.subcore_barrier()
            s_view = s_hbm.at[cb]

            def _g(j, buf, sem):
                pltpu.async_copy(s_view.at[src_v.at[j]], buf, sem)

            def _gw(buf, sem):
                pltpu.make_async_copy(s_view.at[src_v.at[0]], buf, sem).wait()

            def _s(j, buf, sem):
                pltpu.async_copy(buf, acc.at[dst_v.at[j]], sem, add=True)

            def _sw(buf, sem):
                pltpu.make_async_copy(buf, acc.at[dst_v.at[0]], sem).wait()

            for half in range(2):
                pltpu.sync_copy(src_hbm.at[t, pl.ds(half * nch_h, nch_h)],
                                src_v)
                pltpu.sync_copy(dst_hbm.at[t, pl.ds(half * nch_h, nch_h)],
                                dst_v)
                _g(0, buf_a, sem_ga)
                _gw(buf_a, sem_ga)
                _g(1, buf_b, sem_gb)
                _s(0, buf_a, sem_sa)
                _gw(buf_b, sem_gb)
                _sw(buf_a, sem_sa)
                _g(2, buf_a, sem_ga)
                _s(1, buf_b, sem_sb)

                def _pair(i, _):
                    j0 = 2 * i + 2
                    _gw(buf_a, sem_ga)
                    _sw(buf_b, sem_sb)
                    _g(j0 + 1, buf_b, sem_gb)
                    _s(j0, buf_a, sem_sa)
                    _gw(buf_b, sem_gb)
                    _sw(buf_a, sem_sa)
                    _g(j0 + 2, buf_a, sem_ga)
                    _s(j0 + 1, buf_b, sem_sb)
                    return 0

                lax.fori_loop(0, (nch_h - 4) // 2, _pair, 0)
                _gw(buf_a, sem_ga)
                _sw(buf_b, sem_sb)
                _g(nch_h - 1, buf_b, sem_gb)
                _s(nch_h - 2, buf_a, sem_sa)
                _gw(buf_b, sem_gb)
                _sw(buf_a, sem_sa)
                _s(nch_h - 1, buf_b, sem_sb)
                _sw(buf_b, sem_sb)
            plsc.subcore_barrier()
            pltpu.sync_copy(
                acc.at[pl.ds(s * STRIPE, STRIPE)],
                out_hbm.at[out_i].at[pl.ds(s * STRIPE, STRIPE)],
            )

        if split:
            _phase(0, c)
        else:
            for k in range(blocks_per_core):
                cb = c + 2 * k
                _phase(cb, cb)

    return agg


_build_hist_cached = functools.cache(_build_hist)
_build_agg_cached = functools.cache(_build_agg)


def _sc_hist(d16):
    return _build_hist_cached()(d16)


def _sc_agg(nb, w, split, s_blk, src_i, dst_i):
    return _build_agg_cached(nb, w, split)(s_blk, src_i, dst_i)


def _scale_call(v_pad, deg_c):
    br = 512

    def body(v_ref, deg_ref, out_ref):
        dinv = lax.rsqrt(deg_ref[...])
        out_ref[0] = v_ref[...] * dinv

    return pl.pallas_call(
        body,
        grid=(2, N_PAD // br),
        in_specs=[
            pl.BlockSpec((br, 128), lambda cb, r: (r, cb)),
            pl.BlockSpec((br, 1), lambda cb, r: (r, 0)),
        ],
        out_specs=pl.BlockSpec((1, br, 128), lambda cb, r: (cb, r, 0)),
        out_shape=jax.ShapeDtypeStruct((2, N_PAD, 128), F32),
    )(v_pad, deg_c)


def _b_call(e_blk, x, deg_c, w, b_row, relu=True):
    nbk = x.shape[1] // 128
    f_out = w.shape[1]
    br = 256
    bc = min(512, f_out)
    nc = f_out // bc

    def body(e_ref, x_ref, deg_ref, w_ref, b_ref, h_ref, s_ref):
        r = pl.program_id(1)
        deg = deg_ref[...]
        dinv = lax.rsqrt(deg)
        dinv2 = 1.0 / deg
        acc = jnp.zeros((br, bc), F32)
        for k in range(nbk):
            m = dinv * e_ref[k] + dinv2 * x_ref[:, k * 128:(k + 1) * 128]
            acc = acc + lax.dot_general(
                m, w_ref[pl.ds(k * 128, 128), :],
                (((1,), (0,)), ((), ())),
                preferred_element_type=F32)
        z = acc + b_ref[...]
        h = jnp.maximum(z, 0.0) if relu else z
        h_ref[...] = h
        rows = r * br + lax.broadcasted_iota(jnp.int32, (br, bc), 0)
        hm = jnp.where(rows < N, h, 0.0)
        s1 = jnp.sum(hm, axis=0, keepdims=True)
        s2 = jnp.sum(hm * hm, axis=0, keepdims=True)
        sblk = jnp.concatenate([s1, s2, jnp.zeros((6, bc), F32)], axis=0)

        @pl.when(r == 0)
        def _():
            s_ref[...] = sblk

        @pl.when(r > 0)
        def _():
            s_ref[...] = s_ref[...] + sblk

    return pl.pallas_call(
        body,
        grid=(nc, N_PAD // br),
        in_specs=[
            pl.BlockSpec((nbk, br, 128), lambda c, r: (0, r, 0)),
            pl.BlockSpec((br, nbk * 128), lambda c, r: (r, 0)),
            pl.BlockSpec((br, 1), lambda c, r: (r, 0)),
            pl.BlockSpec((nbk * 128, bc), lambda c, r: (0, c)),
            pl.BlockSpec((1, bc), lambda c, r: (0, c)),
        ],
        out_specs=[
            pl.BlockSpec((br, bc), lambda c, r: (r, c)),
            pl.BlockSpec((8, bc), lambda c, r: (0, c)),
        ],
        out_shape=[
            jax.ShapeDtypeStruct((N_PAD, f_out), F32),
            jax.ShapeDtypeStruct((8, f_out), F32),
        ],
    )(e_blk, x, deg_c, w, b_row)


def _c_call(h, s, deg_c, a_row, g_row, be_row, emit_s=True):
    f = h.shape[1]
    br = 512

    def body(h_ref, s_ref, deg_ref, a_ref, g_ref, be_ref, *outs):
        s1 = s_ref[0:1, :]
        s2 = s_ref[1:2, :]
        al = a_ref[...]
        m = s1 * (1.0 / N)
        var = s2 * (1.0 / N) - m * m * al * (2.0 - al)
        inv = lax.rsqrt(var + 1e-5)
        xn = (h_ref[...] - al * m) * inv * g_ref[...] + be_ref[...]
        outs[0][...] = xn
        if emit_s:
            outs[1][0] = xn * lax.rsqrt(deg_ref[...])

    out_specs = [pl.BlockSpec((br, 128), lambda c, r: (r, c))]
    out_shape = [jax.ShapeDtypeStruct((N_PAD, f), F32)]
    if emit_s:
        out_specs.append(pl.BlockSpec((1, br, 128), lambda c, r: (c, r, 0)))
        out_shape.append(jax.ShapeDtypeStruct((f // 128, N_PAD, 128), F32))

    res = pl.pallas_call(
        body,
        grid=(f // 128, N_PAD // br),
        in_specs=[
            pl.BlockSpec((br, 128), lambda c, r: (r, c)),
            pl.BlockSpec((8, 128), lambda c, r: (0, c)),
            pl.BlockSpec((br, 1), lambda c, r: (r, 0)),
            pl.BlockSpec((1, 128), lambda c, r: (0, c)),
            pl.BlockSpec((1, 128), lambda c, r: (0, c)),
            pl.BlockSpec((1, 128), lambda c, r: (0, c)),
        ],
        out_specs=out_specs,
        out_shape=out_shape,
    )(h, s, deg_c, a_row, g_row, be_row)
    return res if emit_s else (res[0],)


def _b4_call(x3, deg_c, wc):
    nbk = x3.shape[1] // 128
    br = 256

    def body(x_ref, deg_ref, w_ref, z_ref, s4_ref):
        acc = jnp.zeros((br, 64), F32)
        for k in range(nbk):
            acc = acc + lax.dot_general(
                x_ref[:, k * 128:(k + 1) * 128],
                w_ref[pl.ds(k * 128, 128), :],
                (((1,), (0,)), ((), ())),
                preferred_element_type=F32)
        z_ref[...] = acc
        s4_ref[0] = jnp.concatenate(
            [acc * lax.rsqrt(deg_ref[...]), jnp.zeros((br, 64), F32)], axis=1)

    return pl.pallas_call(
        body,
        grid=(N_PAD // br,),
        in_specs=[
            pl.BlockSpec((br, nbk * 128), lambda r: (r, 0)),
            pl.BlockSpec((br, 1), lambda r: (r, 0)),
            pl.BlockSpec((nbk * 128, 64), lambda r: (0, 0)),
        ],
        out_specs=[
            pl.BlockSpec((br, 64), lambda r: (r, 0)),
            pl.BlockSpec((1, br, 128), lambda r: (0, r, 0)),
        ],
        out_shape=[
            jax.ShapeDtypeStruct((N_PAD, 64), F32),
            jax.ShapeDtypeStruct((1, N_PAD, 128), F32),
        ],
    )(x3, deg_c, wc)


def _d_call(e4, z4, deg_c, bc_row):
    br = 400

    def body(e_ref, z_ref, deg_ref, b_ref, lg_ref, sm_ref):
        deg = deg_ref[...]
        dinv = lax.rsqrt(deg)
        dinv2 = 1.0 / deg
        lg = (dinv * (e_ref[0, :, :64] + e_ref[1, :, :64])
              + dinv2 * z_ref[...] + b_ref[...])
        lg_ref[...] = lg
        mx = jnp.max(lg, axis=1, keepdims=True)
        p = jnp.exp(lg - mx)
        sm_ref[...] = p / jnp.sum(p, axis=1, keepdims=True)

    return pl.pallas_call(
        body,
        grid=(N // br,),
        in_specs=[
            pl.BlockSpec((2, br, 128), lambda r: (0, r, 0)),
            pl.BlockSpec((br, 64), lambda r: (r, 0)),
            pl.BlockSpec((br, 1), lambda r: (r, 0)),
            pl.BlockSpec((1, 64), lambda r: (0, 0)),
        ],
        out_specs=[
            pl.BlockSpec((br, 64), lambda r: (r, 0)),
            pl.BlockSpec((br, 64), lambda r: (r, 0)),
        ],
        out_shape=[
            jax.ShapeDtypeStruct((N, 64), F32),
            jax.ShapeDtypeStruct((N, 64), F32),
        ],
    )(e4, z4, deg_c, bc_row)


def kernel(v, edges, W1, b1, W2, b2, W3, b3, Wc, bc,
           gn1_alpha, gn1_weight, gn1_bias,
           gn2_alpha, gn2_weight, gn2_bias,
           gn3_alpha, gn3_weight, gn3_bias):
    src = edges[0].astype(jnp.int32)
    dst = edges[1].astype(jnp.int32)
    src_p = jnp.concatenate([src, jnp.zeros((E_PAD - E,), jnp.int32)])
    dst_p = jnp.concatenate([dst, jnp.full((E_PAD - E,), DUMMY_DST, jnp.int32)])
    s16 = src_p.reshape(16, E_PAD // (16 * 64), 64)
    d16 = dst_p.reshape(16, E_PAD // (16 * 64), 64)
    s32 = src_p.reshape(32, E_PAD // (32 * 64), 64)
    d32 = dst_p.reshape(32, E_PAD // (32 * 64), 64)
    v_pad = jnp.pad(v, ((0, N_PAD - N), (0, 0)))

    deg = _sc_hist(d16)
    deg_c = deg.reshape(N_PAD, 1)

    s0 = _scale_call(v_pad, deg_c)
    e1 = _sc_agg(2, 128, False, s0, s16, d16)
    h1, sums1 = _b_call(e1, v_pad, deg_c, W1, b1.reshape(1, -1))
    x1, s1 = _c_call(h1, sums1, deg_c, gn1_alpha.reshape(1, -1),
                     gn1_weight.reshape(1, -1), gn1_bias.reshape(1, -1))

    e2 = _sc_agg(4, 128, False, s1, s16, d16)
    h2, sums2 = _b_call(e2, x1, deg_c, W2, b2.reshape(1, -1))
    x2, s2 = _c_call(h2, sums2, deg_c, gn2_alpha.reshape(1, -1),
                     gn2_weight.reshape(1, -1), gn2_bias.reshape(1, -1))

    e3 = _sc_agg(8, 128, False, s2, s16, d16)
    h3, sums3 = _b_call(e3, x2, deg_c, W3, b3.reshape(1, -1))
    (x3,) = _c_call(h3, sums3, deg_c, gn3_alpha.reshape(1, -1),
                    gn3_weight.reshape(1, -1), gn3_bias.reshape(1, -1),
                    emit_s=False)

    z4, s4 = _b4_call(x3, deg_c, Wc)
    e4 = _sc_agg(1, 128, True, s4, s32, d32)
    logits, sm = _d_call(e4, z4, deg_c, bc.reshape(1, -1))
    return (logits, sm)


# speedup vs baseline: 4.6492x; 1.0852x over previous
import functools

import jax
import jax.numpy as jnp
from jax import lax
from jax.experimental import pallas as pl
from jax.experimental.pallas import tpu as pltpu
from jax.experimental.pallas import tpu_sc as plsc

N = 10000
E = 160000
N_PAD = 10240
E_PAD = 163840
DUMMY_DST = 10008
F32 = jnp.float32

_MESH = functools.partial(
    plsc.VectorSubcoreMesh,
    core_axis_name="c",
    subcore_axis_name="s",
    num_cores=2,
    num_subcores=16,
)
STRIPE = N_PAD // 16


def _build_hist():
    nch = E_PAD // (16 * 64)

    @functools.partial(
        pl.kernel,
        out_type=jax.ShapeDtypeStruct((N_PAD,), F32),
        mesh=_MESH(),
        compiler_params=pltpu.CompilerParams(needs_layout_passes=False),
        scratch_types=[
            pltpu.VMEM((nch, 64), jnp.int32),
            pltpu.VMEM((N_PAD,), F32),
            pltpu.VMEM((16, STRIPE), F32),
            pltpu.VMEM((STRIPE,), F32),
            pltpu.VMEM_SHARED((16, N_PAD), F32),
        ],
    )
    def hist(dst_hbm, out_hbm, idx_v, hist_v, loc2, degl, sh):
        s = lax.axis_index("s")
        pltpu.sync_copy(dst_hbm.at[s], idx_v)

        def _zero(i, _):
            hist_v[pl.ds(i * 16, 16)] = jnp.zeros((16,), F32)
            return 0

        lax.fori_loop(0, N_PAD // 16, _zero, 0)
        ones = jnp.ones((16,), F32)

        def _acc(i, _):
            idx = idx_v[i // 4, pl.ds((i % 4) * 16, 16)]
            plsc.addupdate_scatter(hist_v, [idx], ones)
            return 0

        lax.fori_loop(0, (nch * 64) // 16, _acc, 0)
        pltpu.sync_copy(hist_v, sh.at[s])
        plsc.subcore_barrier()
        for r in range(16):
            pltpu.sync_copy(sh.at[r, pl.ds(s * STRIPE, STRIPE)], loc2.at[r])

        def _merge(j, _):
            sl = pl.ds(j * 16, 16)
            a = loc2[0, sl]
            for r in range(1, 16):
                a = a + loc2[r, sl]
            degl[sl] = a + 1.0
            return 0

        lax.fori_loop(0, STRIPE // 16, _merge, 0)
        pltpu.sync_copy(degl, out_hbm.at[pl.ds(s * STRIPE, STRIPE)])

    return hist


def _build_agg(nb, w, split):
    ntile = 32 if split else 16
    chunk = 128
    nch = E_PAD // (ntile * chunk)
    nch_h = nch // 4
    nb_out = 2 if split else nb
    blocks_per_core = 1 if split else nb // 2

    @functools.partial(
        pl.kernel,
        out_type=jax.ShapeDtypeStruct((nb_out, N_PAD, w), F32),
        mesh=_MESH(),
        compiler_params=pltpu.CompilerParams(needs_layout_passes=False),
        scratch_types=[
            pltpu.VMEM((nch_h, chunk), jnp.int32),
            pltpu.VMEM((nch_h, chunk), jnp.int32),
            pltpu.VMEM((chunk, w), F32),
            pltpu.VMEM((chunk, w), F32),
            pltpu.VMEM_SHARED((N_PAD, w), F32),
            pltpu.SemaphoreType.DMA,
            pltpu.SemaphoreType.DMA,
            pltpu.SemaphoreType.DMA,
            pltpu.SemaphoreType.DMA,
        ],
    )
    def agg(s_hbm, src_hbm, dst_hbm, out_hbm, src_v, dst_v, buf_a, buf_b,
            acc, sem_ga, sem_gb, sem_sa, sem_sb):
        c = lax.axis_index("c")
        s = lax.axis_index("s")
        t = s * 2 + c if split else s

        def _phase(cb, out_i):
            def _zero(i, _):
                buf_a[i // (w // 16), pl.ds((i % (w // 16)) * 16, 16)] = (
                    jnp.zeros((16,), F32))
                buf_b[i // (w // 16), pl.ds((i % (w // 16)) * 16, 16)] = (
                    jnp.zeros((16,), F32))
                return 0

            lax.fori_loop(0, (chunk * w) // 16, _zero, 0)
            for kz in range(STRIPE // chunk):
                pltpu.sync_copy(
                    buf_a, acc.at[pl.ds(s * STRIPE + kz * chunk, chunk)])
            plsc.subcore_barrier()
            s_view = s_hbm.at[cb]

            def _g(j, buf, sem):
                pltpu.async_copy(s_view.at[src_v.at[j]], buf, sem)

            def _gw(buf, sem):
                pltpu.make_async_copy(s_view.at[src_v.at[0]], buf, sem).wait()

            def _s(j, buf, sem):
                pltpu.async_copy(buf, acc.at[dst_v.at[j]], sem, add=True)

            def _sw(buf, sem):
                pltpu.make_async_copy(buf, acc.at[dst_v.at[0]], sem).wait()

            for half in range(4):
                pltpu.sync_copy(src_hbm.at[t, half], src_v)
                pltpu.sync_copy(dst_hbm.at[t, half], dst_v)
                _g(0, buf_a, sem_ga)
                _gw(buf_a, sem_ga)
                _g(1, buf_b, sem_gb)
                _s(0, buf_a, sem_sa)
                _gw(buf_b, sem_gb)
                _sw(buf_a, sem_sa)
                _g(2, buf_a, sem_ga)
                _s(1, buf_b, sem_sb)

                def _pair(i, _):
                    j0 = 2 * i + 2
                    _gw(buf_a, sem_ga)
                    _sw(buf_b, sem_sb)
                    _g(j0 + 1, buf_b, sem_gb)
                    _s(j0, buf_a, sem_sa)
                    _gw(buf_b, sem_gb)
                    _sw(buf_a, sem_sa)
                    _g(j0 + 2, buf_a, sem_ga)
                    _s(j0 + 1, buf_b, sem_sb)
                    return 0

                lax.fori_loop(0, (nch_h - 4) // 2, _pair, 0)
                _gw(buf_a, sem_ga)
                _sw(buf_b, sem_sb)
                _g(nch_h - 1, buf_b, sem_gb)
                _s(nch_h - 2, buf_a, sem_sa)
                _gw(buf_b, sem_gb)
                _sw(buf_a, sem_sa)
                _s(nch_h - 1, buf_b, sem_sb)
                _sw(buf_b, sem_sb)
            plsc.subcore_barrier()
            pltpu.sync_copy(
                acc.at[pl.ds(s * STRIPE, STRIPE)],
                out_hbm.at[out_i].at[pl.ds(s * STRIPE, STRIPE)],
            )

        if split:
            _phase(0, c)
        else:
            for k in range(blocks_per_core):
                cb = c + 2 * k
                _phase(cb, cb)

    return agg


_build_hist_cached = functools.cache(_build_hist)
_build_agg_cached = functools.cache(_build_agg)


def _sc_hist(d16):
    return _build_hist_cached()(d16)


def _sc_agg(nb, w, split, s_blk, src_i, dst_i):
    return _build_agg_cached(nb, w, split)(s_blk, src_i, dst_i)


def _scale_call(v_pad, deg_c):
    br = 512

    def body(v_ref, deg_ref, out_ref):
        dinv = lax.rsqrt(deg_ref[...])
        out_ref[0] = v_ref[...] * dinv

    return pl.pallas_call(
        body,
        grid=(2, N_PAD // br),
        in_specs=[
            pl.BlockSpec((br, 128), lambda cb, r: (r, cb)),
            pl.BlockSpec((br, 1), lambda cb, r: (r, 0)),
        ],
        out_specs=pl.BlockSpec((1, br, 128), lambda cb, r: (cb, r, 0)),
        out_shape=jax.ShapeDtypeStruct((2, N_PAD, 128), F32),
    )(v_pad, deg_c)


def _b_call(e_blk, x, deg_c, w, b_row, relu=True):
    nbk = x.shape[1] // 128
    f_out = w.shape[1]
    br = 256
    bc = min(512, f_out)
    nc = f_out // bc

    def body(e_ref, x_ref, deg_ref, w_ref, b_ref, h_ref, s_ref):
        r = pl.program_id(1)
        deg = deg_ref[...]
        dinv = lax.rsqrt(deg)
        dinv2 = 1.0 / deg
        acc = jnp.zeros((br, bc), F32)
        for k in range(nbk):
            m = dinv * e_ref[k] + dinv2 * x_ref[:, k * 128:(k + 1) * 128]
            acc = acc + lax.dot_general(
                m, w_ref[pl.ds(k * 128, 128), :],
                (((1,), (0,)), ((), ())),
                preferred_element_type=F32)
        z = acc + b_ref[...]
        h = jnp.maximum(z, 0.0) if relu else z
        h_ref[...] = h
        rows = r * br + lax.broadcasted_iota(jnp.int32, (br, bc), 0)
        hm = jnp.where(rows < N, h, 0.0)
        s1 = jnp.sum(hm, axis=0, keepdims=True)
        s2 = jnp.sum(hm * hm, axis=0, keepdims=True)
        sblk = jnp.concatenate([s1, s2, jnp.zeros((6, bc), F32)], axis=0)

        @pl.when(r == 0)
        def _():
            s_ref[...] = sblk

        @pl.when(r > 0)
        def _():
            s_ref[...] = s_ref[...] + sblk

    return pl.pallas_call(
        body,
        grid=(nc, N_PAD // br),
        in_specs=[
            pl.BlockSpec((nbk, br, 128), lambda c, r: (0, r, 0)),
            pl.BlockSpec((br, nbk * 128), lambda c, r: (r, 0)),
            pl.BlockSpec((br, 1), lambda c, r: (r, 0)),
            pl.BlockSpec((nbk * 128, bc), lambda c, r: (0, c)),
            pl.BlockSpec((1, bc), lambda c, r: (0, c)),
        ],
        out_specs=[
            pl.BlockSpec((br, bc), lambda c, r: (r, c)),
            pl.BlockSpec((8, bc), lambda c, r: (0, c)),
        ],
        out_shape=[
            jax.ShapeDtypeStruct((N_PAD, f_out), F32),
            jax.ShapeDtypeStruct((8, f_out), F32),
        ],
    )(e_blk, x, deg_c, w, b_row)


def _c_call(h, s, deg_c, a_row, g_row, be_row, emit_s=True):
    f = h.shape[1]
    br = 512

    def body(h_ref, s_ref, deg_ref, a_ref, g_ref, be_ref, *outs):
        s1 = s_ref[0:1, :]
        s2 = s_ref[1:2, :]
        al = a_ref[...]
        m = s1 * (1.0 / N)
        var = s2 * (1.0 / N) - m * m * al * (2.0 - al)
        inv = lax.rsqrt(var + 1e-5)
        xn = (h_ref[...] - al * m) * inv * g_ref[...] + be_ref[...]
        outs[0][...] = xn
        if emit_s:
            outs[1][0] = xn * lax.rsqrt(deg_ref[...])

    out_specs = [pl.BlockSpec((br, 128), lambda c, r: (r, c))]
    out_shape = [jax.ShapeDtypeStruct((N_PAD, f), F32)]
    if emit_s:
        out_specs.append(pl.BlockSpec((1, br, 128), lambda c, r: (c, r, 0)))
        out_shape.append(jax.ShapeDtypeStruct((f // 128, N_PAD, 128), F32))

    res = pl.pallas_call(
        body,
        grid=(f // 128, N_PAD // br),
        in_specs=[
            pl.BlockSpec((br, 128), lambda c, r: (r, c)),
            pl.BlockSpec((8, 128), lambda c, r: (0, c)),
            pl.BlockSpec((br, 1), lambda c, r: (r, 0)),
            pl.BlockSpec((1, 128), lambda c, r: (0, c)),
            pl.BlockSpec((1, 128), lambda c, r: (0, c)),
            pl.BlockSpec((1, 128), lambda c, r: (0, c)),
        ],
        out_specs=out_specs,
        out_shape=out_shape,
    )(h, s, deg_c, a_row, g_row, be_row)
    return res if emit_s else (res[0],)


def _b4_call(x3, deg_c, wc):
    nbk = x3.shape[1] // 128
    br = 256

    def body(x_ref, deg_ref, w_ref, z_ref, s4_ref):
        acc = jnp.zeros((br, 64), F32)
        for k in range(nbk):
            acc = acc + lax.dot_general(
                x_ref[:, k * 128:(k + 1) * 128],
                w_ref[pl.ds(k * 128, 128), :],
                (((1,), (0,)), ((), ())),
                preferred_element_type=F32)
        z_ref[...] = acc
        s4_ref[0] = jnp.concatenate(
            [acc * lax.rsqrt(deg_ref[...]), jnp.zeros((br, 64), F32)], axis=1)

    return pl.pallas_call(
        body,
        grid=(N_PAD // br,),
        in_specs=[
            pl.BlockSpec((br, nbk * 128), lambda r: (r, 0)),
            pl.BlockSpec((br, 1), lambda r: (r, 0)),
            pl.BlockSpec((nbk * 128, 64), lambda r: (0, 0)),
        ],
        out_specs=[
            pl.BlockSpec((br, 64), lambda r: (r, 0)),
            pl.BlockSpec((1, br, 128), lambda r: (0, r, 0)),
        ],
        out_shape=[
            jax.ShapeDtypeStruct((N_PAD, 64), F32),
            jax.ShapeDtypeStruct((1, N_PAD, 128), F32),
        ],
    )(x3, deg_c, wc)


def _d_call(e4, z4, deg_c, bc_row):
    br = 400

    def body(e_ref, z_ref, deg_ref, b_ref, lg_ref, sm_ref):
        deg = deg_ref[...]
        dinv = lax.rsqrt(deg)
        dinv2 = 1.0 / deg
        lg = (dinv * (e_ref[0, :, :64] + e_ref[1, :, :64])
              + dinv2 * z_ref[...] + b_ref[...])
        lg_ref[...] = lg
        mx = jnp.max(lg, axis=1, keepdims=True)
        p = jnp.exp(lg - mx)
        sm_ref[...] = p / jnp.sum(p, axis=1, keepdims=True)

    return pl.pallas_call(
        body,
        grid=(N // br,),
        in_specs=[
            pl.BlockSpec((2, br, 128), lambda r: (0, r, 0)),
            pl.BlockSpec((br, 64), lambda r: (r, 0)),
            pl.BlockSpec((br, 1), lambda r: (r, 0)),
            pl.BlockSpec((1, 64), lambda r: (0, 0)),
        ],
        out_specs=[
            pl.BlockSpec((br, 64), lambda r: (r, 0)),
            pl.BlockSpec((br, 64), lambda r: (r, 0)),
        ],
        out_shape=[
            jax.ShapeDtypeStruct((N, 64), F32),
            jax.ShapeDtypeStruct((N, 64), F32),
        ],
    )(e4, z4, deg_c, bc_row)


def kernel(v, edges, W1, b1, W2, b2, W3, b3, Wc, bc,
           gn1_alpha, gn1_weight, gn1_bias,
           gn2_alpha, gn2_weight, gn2_bias,
           gn3_alpha, gn3_weight, gn3_bias):
    src = edges[0].astype(jnp.int32)
    dst = edges[1].astype(jnp.int32)
    src_p = jnp.concatenate([src, jnp.zeros((E_PAD - E,), jnp.int32)])
    dst_p = jnp.concatenate([dst, jnp.full((E_PAD - E,), DUMMY_DST, jnp.int32)])
    d16h = dst_p.reshape(16, E_PAD // (16 * 64), 64)
    s16 = src_p.reshape(16, 4, E_PAD // (16 * 4 * 128), 128)
    d16 = dst_p.reshape(16, 4, E_PAD // (16 * 4 * 128), 128)
    s32 = src_p.reshape(32, 4, E_PAD // (32 * 4 * 128), 128)
    d32 = dst_p.reshape(32, 4, E_PAD // (32 * 4 * 128), 128)
    v_pad = jnp.pad(v, ((0, N_PAD - N), (0, 0)))

    deg = _sc_hist(d16h)
    deg_c = deg.reshape(N_PAD, 1)

    s0 = _scale_call(v_pad, deg_c)
    e1 = _sc_agg(2, 128, False, s0, s16, d16)
    h1, sums1 = _b_call(e1, v_pad, deg_c, W1, b1.reshape(1, -1))
    x1, s1 = _c_call(h1, sums1, deg_c, gn1_alpha.reshape(1, -1),
                     gn1_weight.reshape(1, -1), gn1_bias.reshape(1, -1))

    e2 = _sc_agg(4, 128, False, s1, s16, d16)
    h2, sums2 = _b_call(e2, x1, deg_c, W2, b2.reshape(1, -1))
    x2, s2 = _c_call(h2, sums2, deg_c, gn2_alpha.reshape(1, -1),
                     gn2_weight.reshape(1, -1), gn2_bias.reshape(1, -1))

    e3 = _sc_agg(8, 128, False, s2, s16, d16)
    h3, sums3 = _b_call(e3, x2, deg_c, W3, b3.reshape(1, -1))
    (x3,) = _c_call(h3, sums3, deg_c, gn3_alpha.reshape(1, -1),
                    gn3_weight.reshape(1, -1), gn3_bias.reshape(1, -1),
                    emit_s=False)

    z4, s4 = _b4_call(x3, deg_c, Wc)
    e4 = _sc_agg(1, 128, True, s4, s32, d32)
    logits, sm = _d_call(e4, z4, deg_c, bc.reshape(1, -1))
    return (logits, sm)
